# initial kernel scaffold (unmeasured)
import jax
import jax.numpy as jnp
from jax import lax
from jax.experimental import pallas as pl
from jax.experimental.pallas import tpu as pltpu

NZ = 4
B, S, H, Dh, Dr = 2, 512, 16, 128, 32
D = 2048
DCS = 512 // NZ
BS = B * S
SCALE = (Dh + Dr) ** -0.5
BF16 = jnp.bfloat16
F32 = jnp.float32


def _body(x_ref, wdkv_ref, wuk_ref, wuv_ref, wq_ref, wqr_ref, wkr_ref, wo_ref,
          out_ref,
          c_gath, w_gath, q_buf, qr_buf, kr_buf, k_buf, v_buf, o_buf,
          send_sems, recv_sems):
    my_x = lax.axis_index("x")
    my_y = lax.axis_index("y")
    my_z = lax.axis_index("z")

    barrier = pltpu.get_barrier_semaphore()
    for d in range(1, NZ):
        pl.semaphore_signal(
            barrier, inc=1,
            device_id=(my_x, my_y, lax.rem(my_z + d, NZ)),
            device_id_type=pl.DeviceIdType.MESH,
        )
    pl.semaphore_wait(barrier, NZ - 1)

    xb = x_ref[...].reshape(BS, D).astype(BF16)

    w_gath[0, :DCS, :] = wuk_ref[...].astype(BF16)
    w_gath[0, DCS:, :] = wuv_ref[...].astype(BF16)
    c_gath[0, :, :] = jnp.dot(
        xb, wdkv_ref[...].astype(BF16), preferred_element_type=F32
    ).astype(BF16)

    rdmas = []
    for d in range(1, NZ):
        tz = lax.rem(my_z + d, NZ)
        for j, buf in enumerate((w_gath, c_gath)):
            i = (d - 1) * 2 + j
            r = pltpu.make_async_remote_copy(
                src_ref=buf.at[0],
                dst_ref=buf.at[d],
                send_sem=send_sems.at[i],
                recv_sem=recv_sems.at[i],
                device_id=(my_x, my_y, tz),
                device_id_type=pl.DeviceIdType.MESH,
            )
            r.start()
            rdmas.append(r)

    q_buf[...] = jnp.dot(
        xb, wq_ref[...].astype(BF16), preferred_element_type=F32
    ).astype(BF16)
    qr_buf[...] = jnp.dot(
        xb, wqr_ref[...].astype(BF16), preferred_element_type=F32
    ).astype(BF16)
    kr_buf[...] = jnp.dot(
        xb, wkr_ref[...].astype(BF16), preferred_element_type=F32
    ).astype(BF16)

    for r in rdmas:
        r.wait_recv()
    k_acc = jnp.zeros((BS, D), F32)
    v_acc = jnp.zeros((BS, D), F32)
    for d in range(NZ):
        c_d = c_gath[d, :, :]
        k_acc += jnp.dot(c_d, w_gath[d, :DCS, :], preferred_element_type=F32)
        v_acc += jnp.dot(c_d, w_gath[d, DCS:, :], preferred_element_type=F32)
    k_buf[...] = k_acc.astype(BF16)
    v_buf[...] = v_acc.astype(BF16)

    for b in range(B):
        r0 = b * S
        kr = kr_buf[r0:r0 + S, :]
        for h in range(H):
            c0 = h * Dh
            q = q_buf[r0:r0 + S, c0:c0 + Dh]
            k = k_buf[r0:r0 + S, c0:c0 + Dh]
            v = v_buf[r0:r0 + S, c0:c0 + Dh]
            qr = qr_buf[r0:r0 + S, h * Dr:(h + 1) * Dr]
            s = lax.dot_general(q, k, (((1,), (1,)), ((), ())),
                                preferred_element_type=F32)
            s += lax.dot_general(qr, kr, (((1,), (1,)), ((), ())),
                                 preferred_element_type=F32)
            s *= SCALE
            m = jnp.max(s, axis=1, keepdims=True)
            p = jnp.exp(s - m)
            p = p / jnp.sum(p, axis=1, keepdims=True)
            o = lax.dot_general(p.astype(BF16), v, (((1,), (0,)), ((), ())),
                                preferred_element_type=F32)
            o_buf[r0:r0 + S, c0:c0 + Dh] = o.astype(BF16)

    out = jnp.dot(o_buf[...], wo_ref[...].astype(BF16),
                  preferred_element_type=F32)
    out_ref[...] = out.reshape(B, S, D)

    for r in rdmas:
        r.wait_send()


def kernel(x, Wdkv, Wuk, Wuv, Wq, Wqr, Wkr, Wo):
    return pl.pallas_call(
        _body,
        out_shape=jax.ShapeDtypeStruct((B, S, D), F32),
        in_specs=[pl.BlockSpec(memory_space=pltpu.VMEM)] * 8,
        out_specs=pl.BlockSpec(memory_space=pltpu.VMEM),
        scratch_shapes=[
            pltpu.VMEM((NZ, BS, DCS), BF16),
            pltpu.VMEM((NZ, 2 * DCS, D), BF16),
            pltpu.VMEM((BS, D), BF16),
            pltpu.VMEM((BS, H * Dr), BF16),
            pltpu.VMEM((BS, Dr), BF16),
            pltpu.VMEM((BS, D), BF16),
            pltpu.VMEM((BS, D), BF16),
            pltpu.VMEM((BS, D), BF16),
            pltpu.SemaphoreType.DMA((2 * (NZ - 1),)),
            pltpu.SemaphoreType.DMA((2 * (NZ - 1),)),
        ],
        compiler_params=pltpu.CompilerParams(collective_id=0),
    )(x, Wdkv, Wuk, Wuv, Wq, Wqr, Wkr, Wo)


# baseline (device time: 129040 ns/iter reference)
import jax
import jax.numpy as jnp
from jax import lax
from jax.experimental import pallas as pl
from jax.experimental.pallas import tpu as pltpu

NZ = 4
B, S, H, Dh, Dr = 2, 512, 16, 128, 32
D = 2048
DCS = 512 // NZ
BS = B * S
NB = 4
BD = D // NB
SCALE = (Dh + Dr) ** -0.5
BF16 = jnp.bfloat16
F32 = jnp.float32


def _body(x_ref, wdkv_ref, wuk_ref, wuv_ref, wq_ref, wqr_ref, wkr_ref, wo_ref,
          out_ref,
          c_gath, w_gath, q_buf, qr_buf, kr_buf, o_buf, wtile,
          send_sems, recv_sems, cp_sems):
    my_x = lax.axis_index("x")
    my_y = lax.axis_index("y")
    my_z = lax.axis_index("z")

    barrier = pltpu.get_barrier_semaphore()
    for d in range(1, NZ):
        pl.semaphore_signal(
            barrier, inc=1,
            device_id=(my_x, my_y, lax.rem(my_z + d, NZ)),
            device_id_type=pl.DeviceIdType.MESH,
        )
    pl.semaphore_wait(barrier, NZ - 1)

    xb = x_ref[...].reshape(BS, D)

    w_gath[0, :DCS, :] = wuk_ref[...]
    w_gath[0, DCS:, :] = wuv_ref[...]
    c_gath[0, :, :] = jnp.dot(
        xb, wdkv_ref[...], preferred_element_type=F32
    ).astype(BF16)

    rdmas = []
    for d in range(1, NZ):
        tz = lax.rem(my_z + d, NZ)
        for j, buf in enumerate((w_gath, c_gath)):
            i = (d - 1) * 2 + j
            r = pltpu.make_async_remote_copy(
                src_ref=buf.at[0],
                dst_ref=buf.at[d],
                send_sem=send_sems.at[i],
                recv_sem=recv_sems.at[i],
                device_id=(my_x, my_y, tz),
                device_id_type=pl.DeviceIdType.MESH,
            )
            r.start()
            rdmas.append(r)

    kr_buf[...] = jnp.dot(xb, wkr_ref[...], preferred_element_type=F32
                          ).astype(BF16)
    qr_buf[...] = jnp.dot(xb, wqr_ref[...], preferred_element_type=F32
                          ).astype(BF16)

    def stream_matmul(w_hbm, lhs, store):
        copies = []
        for jb in range(NB):
            copies.append(pltpu.make_async_copy(
                w_hbm.at[:, jb * BD:(jb + 1) * BD],
                wtile.at[jb % 2],
                cp_sems.at[jb % 2],
            ))
        copies[0].start()
        for jb in range(NB):
            if jb + 1 < NB:
                copies[jb + 1].start()
            copies[jb].wait()
            store(jb, jnp.dot(lhs, wtile[jb % 2, :, :],
                              preferred_element_type=F32))

    def q_store(jb, val):
        q_buf[:, jb * BD:(jb + 1) * BD] = val.astype(BF16)

    stream_matmul(wq_ref, xb, q_store)

    for r in rdmas:
        r.wait_recv()

    for b in range(B):
        r0 = b * S
        kr = kr_buf[r0:r0 + S, :]
        for h in range(H):
            c0 = h * Dh
            k_bh = jnp.zeros((S, Dh), F32)
            v_bh = jnp.zeros((S, Dh), F32)
            for d in range(NZ):
                c_d = c_gath[d, r0:r0 + S, :]
                k_bh += jnp.dot(c_d, w_gath[d, :DCS, c0:c0 + Dh],
                                preferred_element_type=F32)
                v_bh += jnp.dot(c_d, w_gath[d, DCS:, c0:c0 + Dh],
                                preferred_element_type=F32)
            q = q_buf[r0:r0 + S, c0:c0 + Dh]
            qr = qr_buf[r0:r0 + S, h * Dr:(h + 1) * Dr]
            s = lax.dot_general(q, k_bh.astype(BF16),
                                (((1,), (1,)), ((), ())),
                                preferred_element_type=F32)
            s += lax.dot_general(qr, kr, (((1,), (1,)), ((), ())),
                                 preferred_element_type=F32)
            s *= SCALE
            m = jnp.max(s, axis=1, keepdims=True)
            p = jnp.exp(s - m)
            p = p / jnp.sum(p, axis=1, keepdims=True)
            o = lax.dot_general(p.astype(BF16), v_bh.astype(BF16),
                                (((1,), (0,)), ((), ())),
                                preferred_element_type=F32)
            o_buf[r0:r0 + S, c0:c0 + Dh] = o.astype(BF16)

    ob = o_buf[...]

    def out_store(jb, val):
        out_ref[:, :, jb * BD:(jb + 1) * BD] = val.reshape(B, S, BD)

    stream_matmul(wo_ref, ob, out_store)

    for r in rdmas:
        r.wait_send()


def kernel(x, Wdkv, Wuk, Wuv, Wq, Wqr, Wkr, Wo):
    x, Wdkv, Wuk, Wuv, Wq, Wqr, Wkr, Wo = (
        a.astype(BF16) for a in (x, Wdkv, Wuk, Wuv, Wq, Wqr, Wkr, Wo)
    )
    vmem = pl.BlockSpec(memory_space=pltpu.MemorySpace.VMEM)
    hbm = pl.BlockSpec(memory_space=pltpu.MemorySpace.HBM)
    return pl.pallas_call(
        _body,
        out_shape=jax.ShapeDtypeStruct((B, S, D), F32),
        in_specs=[vmem, vmem, vmem, vmem, hbm, vmem, vmem, hbm],
        out_specs=vmem,
        scratch_shapes=[
            pltpu.VMEM((NZ, BS, DCS), BF16),
            pltpu.VMEM((NZ, 2 * DCS, D), BF16),
            pltpu.VMEM((BS, D), BF16),
            pltpu.VMEM((BS, H * Dr), BF16),
            pltpu.VMEM((BS, Dr), BF16),
            pltpu.VMEM((BS, D), BF16),
            pltpu.VMEM((2, D, BD), BF16),
            pltpu.SemaphoreType.DMA((2 * (NZ - 1),)),
            pltpu.SemaphoreType.DMA((2 * (NZ - 1),)),
            pltpu.SemaphoreType.DMA((2,)),
        ],
        compiler_params=pltpu.CompilerParams(collective_id=0),
    )(x, Wdkv, Wuk, Wuv, Wq, Wqr, Wkr, Wo)


# device time: 124999 ns/iter; 1.0323x vs baseline; 1.0323x over previous
import jax
import jax.numpy as jnp
from jax import lax
from jax.experimental import pallas as pl
from jax.experimental.pallas import tpu as pltpu

NZ = 4
B, S, H, Dh, Dr = 2, 512, 16, 128, 32
D = 2048
DCS = 512 // NZ
BS = B * S
NB = 4
BD = D // NB
SCALE = (Dh + Dr) ** -0.5
BF16 = jnp.bfloat16
F32 = jnp.float32


def _body(x_ref, wdkv_ref, wuk_ref, wuv_ref, wq_ref, wqr_ref, wkr_ref, wo_ref,
          out_ref,
          c_gath, w_gath, q_buf, qr_buf, kr_buf, o_buf, wtile,
          send_sems, recv_sems, cp_sems):
    my_x = lax.axis_index("x")
    my_y = lax.axis_index("y")
    my_z = lax.axis_index("z")

    barrier = pltpu.get_barrier_semaphore()
    for d in range(1, NZ):
        pl.semaphore_signal(
            barrier, inc=1,
            device_id=(my_x, my_y, lax.rem(my_z + d, NZ)),
            device_id_type=pl.DeviceIdType.MESH,
        )
    pl.semaphore_wait(barrier, NZ - 1)

    xb = x_ref[...].reshape(BS, D).astype(BF16)

    w_gath[0, :DCS, :] = wuk_ref[...].astype(BF16)
    w_gath[0, DCS:, :] = wuv_ref[...].astype(BF16)
    c_gath[0, :, :] = jnp.dot(
        xb, wdkv_ref[...].astype(BF16), preferred_element_type=F32
    ).astype(BF16)

    rdmas = []
    for d in range(1, NZ):
        tz = lax.rem(my_z + d, NZ)
        for j, buf in enumerate((w_gath, c_gath)):
            i = (d - 1) * 2 + j
            r = pltpu.make_async_remote_copy(
                src_ref=buf.at[0],
                dst_ref=buf.at[d],
                send_sem=send_sems.at[i],
                recv_sem=recv_sems.at[i],
                device_id=(my_x, my_y, tz),
                device_id_type=pl.DeviceIdType.MESH,
            )
            r.start()
            rdmas.append(r)

    kr_buf[...] = jnp.dot(xb, wkr_ref[...].astype(BF16),
                          preferred_element_type=F32).astype(BF16)
    qr_buf[...] = jnp.dot(xb, wqr_ref[...].astype(BF16),
                          preferred_element_type=F32).astype(BF16)

    def stream_matmul(w_hbm, lhs, store):
        copies = []
        for jb in range(NB):
            copies.append(pltpu.make_async_copy(
                w_hbm.at[:, jb * BD:(jb + 1) * BD],
                wtile.at[jb % 2],
                cp_sems.at[jb % 2],
            ))
        copies[0].start()
        for jb in range(NB):
            if jb + 1 < NB:
                copies[jb + 1].start()
            copies[jb].wait()
            store(jb, jnp.dot(lhs, wtile[jb % 2, :, :],
                              preferred_element_type=F32))

    def q_store(jb, val):
        q_buf[:, jb * BD:(jb + 1) * BD] = val.astype(BF16)

    stream_matmul(wq_ref, xb, q_store)

    for r in rdmas:
        r.wait_recv()

    for b in range(B):
        r0 = b * S
        kr = kr_buf[r0:r0 + S, :]
        for h in range(H):
            c0 = h * Dh
            k_bh = jnp.zeros((S, Dh), F32)
            v_bh = jnp.zeros((S, Dh), F32)
            for d in range(NZ):
                c_d = c_gath[d, r0:r0 + S, :]
                k_bh += jnp.dot(c_d, w_gath[d, :DCS, c0:c0 + Dh],
                                preferred_element_type=F32)
                v_bh += jnp.dot(c_d, w_gath[d, DCS:, c0:c0 + Dh],
                                preferred_element_type=F32)
            q = q_buf[r0:r0 + S, c0:c0 + Dh]
            qr = qr_buf[r0:r0 + S, h * Dr:(h + 1) * Dr]
            s = lax.dot_general(q, k_bh.astype(BF16),
                                (((1,), (1,)), ((), ())),
                                preferred_element_type=F32)
            s += lax.dot_general(qr, kr, (((1,), (1,)), ((), ())),
                                 preferred_element_type=F32)
            s *= SCALE
            m = jnp.max(s, axis=1, keepdims=True)
            p = jnp.exp(s - m)
            p = p / jnp.sum(p, axis=1, keepdims=True)
            o = lax.dot_general(p.astype(BF16), v_bh.astype(BF16),
                                (((1,), (0,)), ((), ())),
                                preferred_element_type=F32)
            o_buf[r0:r0 + S, c0:c0 + Dh] = o.astype(BF16)

    ob = o_buf[...]

    def out_store(jb, val):
        out_ref[:, :, jb * BD:(jb + 1) * BD] = val.reshape(B, S, BD)

    stream_matmul(wo_ref, ob, out_store)

    for r in rdmas:
        r.wait_send()


def kernel(x, Wdkv, Wuk, Wuv, Wq, Wqr, Wkr, Wo):
    Wq = Wq.astype(BF16)
    Wo = Wo.astype(BF16)
    vmem = pl.BlockSpec(memory_space=pltpu.MemorySpace.VMEM)
    hbm = pl.BlockSpec(memory_space=pltpu.MemorySpace.HBM)
    return pl.pallas_call(
        _body,
        out_shape=jax.ShapeDtypeStruct((B, S, D), F32),
        in_specs=[vmem, vmem, vmem, vmem, hbm, vmem, vmem, hbm],
        out_specs=vmem,
        scratch_shapes=[
            pltpu.VMEM((NZ, BS, DCS), BF16),
            pltpu.VMEM((NZ, 2 * DCS, D), BF16),
            pltpu.VMEM((BS, D), BF16),
            pltpu.VMEM((BS, H * Dr), BF16),
            pltpu.VMEM((BS, Dr), BF16),
            pltpu.VMEM((BS, D), BF16),
            pltpu.VMEM((2, D, BD), BF16),
            pltpu.SemaphoreType.DMA((2 * (NZ - 1),)),
            pltpu.SemaphoreType.DMA((2 * (NZ - 1),)),
            pltpu.SemaphoreType.DMA((2,)),
        ],
        compiler_params=pltpu.CompilerParams(collective_id=0),
    )(x, Wdkv, Wuk, Wuv, Wq, Wqr, Wkr, Wo)


# device time: 98295 ns/iter; 1.3128x vs baseline; 1.2717x over previous
import jax
import jax.numpy as jnp
from jax import lax
from jax.experimental import pallas as pl
from jax.experimental.pallas import tpu as pltpu

NZ = 4
NP = 4
B, S, H, Dh, Dr = 2, 512, 16, 128, 32
D = 2048
DCS = 512 // NZ
BS = B * S
HL = H // NP
HB = HL * Dh
NBO = 4
BD = D // NBO
SCALE = (Dh + Dr) ** -0.5
BF16 = jnp.bfloat16
F32 = jnp.float32


def _body(x_ref, wdkv_ref, wuk_ref, wuv_ref, wq_ref, wqr_ref, wkr_ref, wo_ref,
          out_ref,
          c_gath, w_cast, w_gath, q_tile, q_own, wqr_tile, qr_own, kr_buf,
          o_own, o_gath, wtile,
          z_send_sems, z_recv_sems, o_send_sems, o_recv_sems, local_sems):
    my_x = lax.axis_index("x")
    my_y = lax.axis_index("y")
    my_z = lax.axis_index("z")
    my_p = my_x * 2 + my_y

    barrier = pltpu.get_barrier_semaphore()
    for d in range(1, NZ):
        pl.semaphore_signal(
            barrier, inc=1,
            device_id=(my_x, my_y, lax.rem(my_z + d, NZ)),
            device_id_type=pl.DeviceIdType.MESH,
        )
    for d in range(1, NP):
        pt = lax.rem(my_p + d, NP)
        pl.semaphore_signal(
            barrier, inc=1,
            device_id=(pt // 2, lax.rem(pt, 2), my_z),
            device_id_type=pl.DeviceIdType.MESH,
        )
    pl.semaphore_wait(barrier, NZ - 1 + NP - 1)

    xb = x_ref[...].reshape(BS, D).astype(BF16)

    w_cast[:DCS, :] = wuk_ref[...].astype(BF16)
    w_cast[DCS:, :] = wuv_ref[...].astype(BF16)
    cp_w = pltpu.make_async_copy(
        w_cast.at[:, pl.ds(my_p * HB, HB)], w_gath.at[0], local_sems.at[0])
    cp_w.start()
    c_gath[0, :, :] = jnp.dot(
        xb, wdkv_ref[...].astype(BF16), preferred_element_type=F32
    ).astype(BF16)
    cp_w.wait()

    z_rdmas = []
    for d in range(1, NZ):
        tz = lax.rem(my_z + d, NZ)
        for j, buf in enumerate((w_gath, c_gath)):
            i = (d - 1) * 2 + j
            r = pltpu.make_async_remote_copy(
                src_ref=buf.at[0],
                dst_ref=buf.at[d],
                send_sem=z_send_sems.at[i],
                recv_sem=z_recv_sems.at[i],
                device_id=(my_x, my_y, tz),
                device_id_type=pl.DeviceIdType.MESH,
            )
            r.start()
            z_rdmas.append(r)

    cp_q = pltpu.make_async_copy(
        wq_ref.at[:, pl.ds(my_p * HB, HB)], q_tile, local_sems.at[1])
    cp_q.start()
    cp_qr = pltpu.make_async_copy(
        wqr_ref.at[:, pl.ds(my_p * HL * Dr, HL * Dr)], wqr_tile,
        local_sems.at[2])
    cp_qr.start()
    kr_buf[...] = jnp.dot(xb, wkr_ref[...].astype(BF16),
                          preferred_element_type=F32).astype(BF16)
    cp_qr.wait()
    qr_own[...] = jnp.dot(xb, wqr_tile[...].astype(BF16),
                          preferred_element_type=F32).astype(BF16)
    cp_q.wait()
    q_own[...] = jnp.dot(xb, q_tile[...],
                         preferred_element_type=F32).astype(BF16)

    for r in z_rdmas:
        r.wait_recv()

    for b in range(B):
        r0 = b * S
        kr = kr_buf[r0:r0 + S, :]
        for hl in range(HL):
            c0 = hl * Dh
            k_bh = jnp.zeros((S, Dh), F32)
            v_bh = jnp.zeros((S, Dh), F32)
            for d in range(NZ):
                c_d = c_gath[d, r0:r0 + S, :]
                k_bh += jnp.dot(c_d, w_gath[d, :DCS, c0:c0 + Dh],
                                preferred_element_type=F32)
                v_bh += jnp.dot(c_d, w_gath[d, DCS:, c0:c0 + Dh],
                                preferred_element_type=F32)
            q = q_own[r0:r0 + S, c0:c0 + Dh]
            qr = qr_own[r0:r0 + S, hl * Dr:(hl + 1) * Dr]
            s = lax.dot_general(q, k_bh.astype(BF16),
                                (((1,), (1,)), ((), ())),
                                preferred_element_type=F32)
            s += lax.dot_general(qr, kr, (((1,), (1,)), ((), ())),
                                 preferred_element_type=F32)
            s *= SCALE
            m = jnp.max(s, axis=1, keepdims=True)
            p = jnp.exp(s - m)
            p = p / jnp.sum(p, axis=1, keepdims=True)
            o = lax.dot_general(p.astype(BF16), v_bh.astype(BF16),
                                (((1,), (0,)), ((), ())),
                                preferred_element_type=F32)
            o_own[r0:r0 + S, c0:c0 + Dh] = o.astype(BF16)

    cp_o = pltpu.make_async_copy(o_own, o_gath.at[my_p], local_sems.at[3])
    cp_o.start()
    o_rdmas = []
    for d in range(1, NP):
        pt = lax.rem(my_p + d, NP)
        r = pltpu.make_async_remote_copy(
            src_ref=o_own,
            dst_ref=o_gath.at[my_p],
            send_sem=o_send_sems.at[d - 1],
            recv_sem=o_recv_sems.at[my_p],
            device_id=(pt // 2, lax.rem(pt, 2), my_z),
            device_id_type=pl.DeviceIdType.MESH,
        )
        r.start()
        o_rdmas.append(r)

    wo_copies = [
        pltpu.make_async_copy(
            wo_ref.at[:, jb * BD:(jb + 1) * BD],
            wtile.at[jb % 2],
            local_sems.at[jb % 2],
        )
        for jb in range(NBO)
    ]
    wo_copies[0].start()

    cp_o.wait()
    for d in range(1, NP):
        sp = lax.rem(my_p + d, NP)
        recv = pltpu.make_async_remote_copy(
            src_ref=o_own,
            dst_ref=o_gath.at[sp],
            send_sem=o_send_sems.at[d - 1],
            recv_sem=o_recv_sems.at[sp],
            device_id=(my_x, my_y, my_z),
            device_id_type=pl.DeviceIdType.MESH,
        )
        recv.wait_recv()

    for jb in range(NBO):
        if jb + 1 < NBO:
            wo_copies[jb + 1].start()
        wo_copies[jb].wait()
        acc = jnp.zeros((BS, BD), F32)
        for q in range(NP):
            acc += jnp.dot(o_gath[q, :, :],
                           wtile[jb % 2, q * HB:(q + 1) * HB, :],
                           preferred_element_type=F32)
        out_ref[:, :, jb * BD:(jb + 1) * BD] = acc.reshape(B, S, BD)

    for r in z_rdmas:
        r.wait_send()
    for r in o_rdmas:
        r.wait_send()


def kernel(x, Wdkv, Wuk, Wuv, Wq, Wqr, Wkr, Wo):
    Wq = Wq.astype(BF16)
    Wo = Wo.astype(BF16)
    vmem = pl.BlockSpec(memory_space=pltpu.MemorySpace.VMEM)
    hbm = pl.BlockSpec(memory_space=pltpu.MemorySpace.HBM)
    return pl.pallas_call(
        _body,
        out_shape=jax.ShapeDtypeStruct((B, S, D), F32),
        in_specs=[vmem, vmem, vmem, vmem, hbm, vmem, vmem, hbm],
        out_specs=vmem,
        scratch_shapes=[
            pltpu.VMEM((NZ, BS, DCS), BF16),
            pltpu.VMEM((2 * DCS, D), BF16),
            pltpu.VMEM((NZ, 2 * DCS, HB), BF16),
            pltpu.VMEM((D, HB), BF16),
            pltpu.VMEM((BS, HB), BF16),
            pltpu.VMEM((D, HL * Dr), F32),
            pltpu.VMEM((BS, HL * Dr), BF16),
            pltpu.VMEM((BS, Dr), BF16),
            pltpu.VMEM((BS, HB), BF16),
            pltpu.VMEM((NP, BS, HB), BF16),
            pltpu.VMEM((2, D, BD), BF16),
            pltpu.SemaphoreType.DMA((2 * (NZ - 1),)),
            pltpu.SemaphoreType.DMA((2 * (NZ - 1),)),
            pltpu.SemaphoreType.DMA((NP - 1,)),
            pltpu.SemaphoreType.DMA((NP,)),
            pltpu.SemaphoreType.DMA((4,)),
        ],
        compiler_params=pltpu.CompilerParams(collective_id=0),
    )(x, Wdkv, Wuk, Wuv, Wq, Wqr, Wkr, Wo)


# device time: 81376 ns/iter; 1.5857x vs baseline; 1.2079x over previous
import jax
import jax.numpy as jnp
from jax import lax
from jax.experimental import pallas as pl
from jax.experimental.pallas import tpu as pltpu

NZ = 4
NP = 4
B, S, H, Dh, Dr = 2, 512, 16, 128, 32
D = 2048
DCS = 512 // NZ
BS = B * S
HL = H // NP
HB = HL * Dh
NBO = 4
BD = D // NBO
SCALE = (Dh + Dr) ** -0.5
BF16 = jnp.bfloat16
F32 = jnp.float32


def _body(x_ref, wdkv_ref, wuk_ref, wuv_ref, wq_ref, wqr_ref, wkr_ref, wo_ref,
          out_ref,
          x_scr, c_gath, w_cast, w_gath, q_tile, q_own, wqr_tile, qr_own,
          kr_buf, o_own, o_gath, wtile,
          z_send_sems, z_recv_sems, o_send_sems, o_recv_sems, local_sems):
    my_x = lax.axis_index("x")
    my_y = lax.axis_index("y")
    my_z = lax.axis_index("z")
    my_p = my_x * 2 + my_y

    cp_x = pltpu.make_async_copy(x_ref, x_scr, local_sems.at[3])
    cp_x.start()

    barrier = pltpu.get_barrier_semaphore()
    for d in range(1, NZ):
        pl.semaphore_signal(
            barrier, inc=1,
            device_id=(my_x, my_y, lax.rem(my_z + d, NZ)),
            device_id_type=pl.DeviceIdType.MESH,
        )
    for d in range(1, NP):
        pt = lax.rem(my_p + d, NP)
        pl.semaphore_signal(
            barrier, inc=1,
            device_id=(pt // 2, lax.rem(pt, 2), my_z),
            device_id_type=pl.DeviceIdType.MESH,
        )
    pl.semaphore_wait(barrier, NZ - 1 + NP - 1)

    cp_x.wait()
    xb = x_scr[...].reshape(BS, D).astype(BF16)

    w_cast[:DCS, :] = wuk_ref[...].astype(BF16)
    w_cast[DCS:, :] = wuv_ref[...].astype(BF16)
    cp_w = pltpu.make_async_copy(
        w_cast.at[:, pl.ds(my_p * HB, HB)], w_gath.at[0], local_sems.at[0])
    cp_w.start()
    c_gath[0, :, :] = jnp.dot(
        xb, wdkv_ref[...].astype(BF16), preferred_element_type=F32
    ).astype(BF16)
    cp_w.wait()

    z_rdmas = []
    for d in range(1, NZ):
        tz = lax.rem(my_z + d, NZ)
        for j, buf in enumerate((w_gath, c_gath)):
            i = (d - 1) * 2 + j
            r = pltpu.make_async_remote_copy(
                src_ref=buf.at[0],
                dst_ref=buf.at[d],
                send_sem=z_send_sems.at[i],
                recv_sem=z_recv_sems.at[i],
                device_id=(my_x, my_y, tz),
                device_id_type=pl.DeviceIdType.MESH,
            )
            r.start()
            z_rdmas.append(r)

    cp_q = pltpu.make_async_copy(
        wq_ref.at[:, pl.ds(my_p * HB, HB)], q_tile, local_sems.at[1])
    cp_q.start()
    cp_qr = pltpu.make_async_copy(
        wqr_ref.at[:, pl.ds(my_p * HL * Dr, HL * Dr)], wqr_tile,
        local_sems.at[2])
    cp_qr.start()
    kr_buf[...] = jnp.dot(xb, wkr_ref[...].astype(BF16),
                          preferred_element_type=F32).astype(BF16)
    cp_qr.wait()
    qr_own[...] = jnp.dot(xb, wqr_tile[...].astype(BF16),
                          preferred_element_type=F32).astype(BF16)
    cp_q.wait()
    q_own[...] = jnp.dot(xb, q_tile[...].astype(BF16),
                         preferred_element_type=F32).astype(BF16)

    for r in z_rdmas:
        r.wait_recv()

    for b in range(B):
        r0 = b * S
        kr = kr_buf[r0:r0 + S, :]
        for hl in range(HL):
            c0 = hl * Dh
            k_bh = jnp.zeros((S, Dh), F32)
            v_bh = jnp.zeros((S, Dh), F32)
            for d in range(NZ):
                c_d = c_gath[d, r0:r0 + S, :]
                k_bh += jnp.dot(c_d, w_gath[d, :DCS, c0:c0 + Dh],
                                preferred_element_type=F32)
                v_bh += jnp.dot(c_d, w_gath[d, DCS:, c0:c0 + Dh],
                                preferred_element_type=F32)
            q = q_own[r0:r0 + S, c0:c0 + Dh]
            qr = qr_own[r0:r0 + S, hl * Dr:(hl + 1) * Dr]
            s = lax.dot_general(q, k_bh.astype(BF16),
                                (((1,), (1,)), ((), ())),
                                preferred_element_type=F32)
            s += lax.dot_general(qr, kr, (((1,), (1,)), ((), ())),
                                 preferred_element_type=F32)
            s *= SCALE
            m = jnp.max(s, axis=1, keepdims=True)
            p = jnp.exp(s - m)
            p = p / jnp.sum(p, axis=1, keepdims=True)
            o = lax.dot_general(p.astype(BF16), v_bh.astype(BF16),
                                (((1,), (0,)), ((), ())),
                                preferred_element_type=F32)
            o_own[r0:r0 + S, c0:c0 + Dh] = o.astype(BF16)

    o_rdmas = []
    for d in range(1, NP):
        pt = lax.rem(my_p + d, NP)
        r = pltpu.make_async_remote_copy(
            src_ref=o_own,
            dst_ref=o_gath.at[d],
            send_sem=o_send_sems.at[d - 1],
            recv_sem=o_recv_sems.at[d - 1],
            device_id=(pt // 2, lax.rem(pt, 2), my_z),
            device_id_type=pl.DeviceIdType.MESH,
        )
        r.start()
        o_rdmas.append(r)

    row_offs = [my_p] + [lax.rem(my_p + NP - i, NP) for i in range(1, NP)]
    wo_copies = [
        pltpu.make_async_copy(
            wo_ref.at[pl.ds(row_offs[i] * HB, HB), :],
            wtile.at[i % 2],
            local_sems.at[i % 2],
        )
        for i in range(NP)
    ]
    wo_copies[0].start()
    for i in range(NP):
        if i + 1 < NP:
            wo_copies[i + 1].start()
        if i > 0:
            o_rdmas[i - 1].wait_recv()
        wo_copies[i].wait()
        lhs = o_own[...] if i == 0 else o_gath[i, :, :]
        for jc in range(2):
            cs = jc * (D // 2)
            contrib = jnp.dot(
                lhs, wtile[i % 2, :, cs:cs + D // 2].astype(BF16),
                preferred_element_type=F32).reshape(B, S, D // 2)
            if i == 0:
                out_ref[:, :, cs:cs + D // 2] = contrib
            else:
                out_ref[:, :, cs:cs + D // 2] = (
                    out_ref[:, :, cs:cs + D // 2] + contrib)

    for r in z_rdmas:
        r.wait_send()
    for r in o_rdmas:
        r.wait_send()


def kernel(x, Wdkv, Wuk, Wuv, Wq, Wqr, Wkr, Wo):
    vmem = pl.BlockSpec(memory_space=pltpu.MemorySpace.VMEM)
    hbm = pl.BlockSpec(memory_space=pltpu.MemorySpace.HBM)
    return pl.pallas_call(
        _body,
        out_shape=jax.ShapeDtypeStruct((B, S, D), F32),
        in_specs=[hbm, vmem, vmem, vmem, hbm, hbm, vmem, hbm],
        out_specs=vmem,
        scratch_shapes=[
            pltpu.VMEM((B, S, D), F32),
            pltpu.VMEM((NZ, BS, DCS), BF16),
            pltpu.VMEM((2 * DCS, D), BF16),
            pltpu.VMEM((NZ, 2 * DCS, HB), BF16),
            pltpu.VMEM((D, HB), F32),
            pltpu.VMEM((BS, HB), BF16),
            pltpu.VMEM((D, HL * Dr), F32),
            pltpu.VMEM((BS, HL * Dr), BF16),
            pltpu.VMEM((BS, Dr), BF16),
            pltpu.VMEM((BS, HB), BF16),
            pltpu.VMEM((NP, BS, HB), BF16),
            pltpu.VMEM((2, HB, D), F32),
            pltpu.SemaphoreType.DMA((2 * (NZ - 1),)),
            pltpu.SemaphoreType.DMA((2 * (NZ - 1),)),
            pltpu.SemaphoreType.DMA((NP - 1,)),
            pltpu.SemaphoreType.DMA((NP - 1,)),
            pltpu.SemaphoreType.DMA((4,)),
        ],
        compiler_params=pltpu.CompilerParams(
            collective_id=0,
            vmem_limit_bytes=63 * 1024 * 1024,
        ),
    )(x, Wdkv, Wuk, Wuv, Wq, Wqr, Wkr, Wo)


# device time: 74702 ns/iter; 1.7274x vs baseline; 1.0893x over previous
import jax
import jax.numpy as jnp
from jax import lax
from jax.experimental import pallas as pl
from jax.experimental.pallas import tpu as pltpu

NZ = 4
NP = 4
B, S, H, Dh, Dr = 2, 512, 16, 128, 32
D = 2048
DCS = 512 // NZ
BS = B * S
HL = H // NP
HB = HL * Dh
NBO = 4
BD = D // NBO
SCALE = (Dh + Dr) ** -0.5
BF16 = jnp.bfloat16
F32 = jnp.float32


def _body(x_ref, wdkv_ref, wuk_ref, wuv_ref, wq_ref, wqr_ref, wkr_ref, wo_ref,
          out_ref,
          x_scr, c_gath, w_cast, w_gath, q_tile, q_own, wqr_tile, qr_own,
          kr_buf, o_own, o_gath, wtile,
          z_send_sems, z_recv_sems, o_send_sems, o_recv_sems, local_sems):
    my_x = lax.axis_index("x")
    my_y = lax.axis_index("y")
    my_z = lax.axis_index("z")
    my_p = my_x * 2 + my_y

    cp_x = pltpu.make_async_copy(x_ref, x_scr, local_sems.at[3])
    cp_x.start()
    cp_q = pltpu.make_async_copy(
        wq_ref.at[:, pl.ds(my_p * HB, HB)], q_tile, local_sems.at[1])
    cp_q.start()
    cp_qr = pltpu.make_async_copy(
        wqr_ref.at[:, pl.ds(my_p * HL * Dr, HL * Dr)], wqr_tile,
        local_sems.at[2])
    cp_qr.start()

    barrier = pltpu.get_barrier_semaphore()
    for d in range(1, NZ):
        pl.semaphore_signal(
            barrier, inc=1,
            device_id=(my_x, my_y, lax.rem(my_z + d, NZ)),
            device_id_type=pl.DeviceIdType.MESH,
        )
    for d in range(1, NP):
        pt = lax.rem(my_p + d, NP)
        pl.semaphore_signal(
            barrier, inc=1,
            device_id=(pt // 2, lax.rem(pt, 2), my_z),
            device_id_type=pl.DeviceIdType.MESH,
        )
    pl.semaphore_wait(barrier, NZ - 1 + NP - 1)

    cp_x.wait()
    xb = x_scr[...].reshape(BS, D).astype(BF16)

    w_cast[:DCS, :] = wuk_ref[...].astype(BF16)
    w_cast[DCS:, :] = wuv_ref[...].astype(BF16)
    cp_w = pltpu.make_async_copy(
        w_cast.at[:, pl.ds(my_p * HB, HB)], w_gath.at[0], local_sems.at[0])
    cp_w.start()
    c_gath[0, :, :] = jnp.dot(
        xb, wdkv_ref[...].astype(BF16), preferred_element_type=F32
    ).astype(BF16)
    cp_w.wait()

    z_rdmas = []
    for d in range(1, NZ):
        tz = lax.rem(my_z + d, NZ)
        for j, buf in enumerate((w_gath, c_gath)):
            i = (d - 1) * 2 + j
            r = pltpu.make_async_remote_copy(
                src_ref=buf.at[0],
                dst_ref=buf.at[d],
                send_sem=z_send_sems.at[i],
                recv_sem=z_recv_sems.at[i],
                device_id=(my_x, my_y, tz),
                device_id_type=pl.DeviceIdType.MESH,
            )
            r.start()
            z_rdmas.append(r)

    kr_buf[...] = jnp.dot(xb, wkr_ref[...].astype(BF16),
                          preferred_element_type=F32).astype(BF16)
    cp_qr.wait()
    qr_own[...] = jnp.dot(xb, wqr_tile[...].astype(BF16),
                          preferred_element_type=F32).astype(BF16)
    cp_q.wait()
    q_own[...] = jnp.dot(xb, q_tile[...].astype(BF16),
                         preferred_element_type=F32).astype(BF16)

    for r in z_rdmas:
        r.wait_recv()

    row_offs = [my_p] + [lax.rem(my_p + NP - i, NP) for i in range(1, NP)]
    wo_copies = [
        pltpu.make_async_copy(
            wo_ref.at[pl.ds(row_offs[i] * HB, HB), :],
            wtile.at[i % 2],
            local_sems.at[i % 2],
        )
        for i in range(NP)
    ]
    wo_copies[0].start()
    wo_copies[1].start()

    o_rdmas = []
    for b in range(B):
        r0 = b * S
        kr = kr_buf[r0:r0 + S, :]
        for hl in range(HL):
            c0 = hl * Dh
            k_bh = jnp.zeros((S, Dh), F32)
            v_bh = jnp.zeros((S, Dh), F32)
            for d in range(NZ):
                c_d = c_gath[d, r0:r0 + S, :]
                k_bh += jnp.dot(c_d, w_gath[d, :DCS, c0:c0 + Dh],
                                preferred_element_type=F32)
                v_bh += jnp.dot(c_d, w_gath[d, DCS:, c0:c0 + Dh],
                                preferred_element_type=F32)
            q = q_own[r0:r0 + S, c0:c0 + Dh]
            qr = qr_own[r0:r0 + S, hl * Dr:(hl + 1) * Dr]
            s = lax.dot_general(q, k_bh.astype(BF16),
                                (((1,), (1,)), ((), ())),
                                preferred_element_type=F32)
            s += lax.dot_general(qr, kr, (((1,), (1,)), ((), ())),
                                 preferred_element_type=F32)
            s *= SCALE
            m = jnp.max(s, axis=1, keepdims=True)
            p = jnp.exp(s - m)
            denom = jnp.sum(p, axis=1, keepdims=True)
            o = lax.dot_general(p.astype(BF16), v_bh.astype(BF16),
                                (((1,), (0,)), ((), ())),
                                preferred_element_type=F32)
            o = o / denom
            o_own[r0:r0 + S, c0:c0 + Dh] = o.astype(BF16)
        for d in range(1, NP):
            pt = lax.rem(my_p + d, NP)
            i = (d - 1) * B + b
            r = pltpu.make_async_remote_copy(
                src_ref=o_own.at[pl.ds(r0, S), :],
                dst_ref=o_gath.at[d, pl.ds(r0, S), :],
                send_sem=o_send_sems.at[i],
                recv_sem=o_recv_sems.at[i],
                device_id=(pt // 2, lax.rem(pt, 2), my_z),
                device_id_type=pl.DeviceIdType.MESH,
            )
            r.start()
            o_rdmas.append(r)

    for i in range(NP):
        if 2 <= i + 1 < NP:
            wo_copies[i + 1].start()
        if i > 0:
            for b in range(B):
                o_rdmas[(i - 1) * B + b].wait_recv()
        wo_copies[i].wait()
        lhs = o_own[...] if i == 0 else o_gath[i, :, :]
        for jc in range(2):
            cs = jc * (D // 2)
            contrib = jnp.dot(
                lhs, wtile[i % 2, :, cs:cs + D // 2].astype(BF16),
                preferred_element_type=F32).reshape(B, S, D // 2)
            if i == 0:
                out_ref[:, :, cs:cs + D // 2] = contrib
            else:
                out_ref[:, :, cs:cs + D // 2] = (
                    out_ref[:, :, cs:cs + D // 2] + contrib)

    for r in z_rdmas:
        r.wait_send()
    for r in o_rdmas:
        r.wait_send()


def kernel(x, Wdkv, Wuk, Wuv, Wq, Wqr, Wkr, Wo):
    vmem = pl.BlockSpec(memory_space=pltpu.MemorySpace.VMEM)
    hbm = pl.BlockSpec(memory_space=pltpu.MemorySpace.HBM)
    return pl.pallas_call(
        _body,
        out_shape=jax.ShapeDtypeStruct((B, S, D), F32),
        in_specs=[hbm, vmem, vmem, vmem, hbm, hbm, vmem, hbm],
        out_specs=vmem,
        scratch_shapes=[
            pltpu.VMEM((B, S, D), F32),
            pltpu.VMEM((NZ, BS, DCS), BF16),
            pltpu.VMEM((2 * DCS, D), BF16),
            pltpu.VMEM((NZ, 2 * DCS, HB), BF16),
            pltpu.VMEM((D, HB), F32),
            pltpu.VMEM((BS, HB), BF16),
            pltpu.VMEM((D, HL * Dr), F32),
            pltpu.VMEM((BS, HL * Dr), BF16),
            pltpu.VMEM((BS, Dr), BF16),
            pltpu.VMEM((BS, HB), BF16),
            pltpu.VMEM((NP, BS, HB), BF16),
            pltpu.VMEM((2, HB, D), F32),
            pltpu.SemaphoreType.DMA((2 * (NZ - 1),)),
            pltpu.SemaphoreType.DMA((2 * (NZ - 1),)),
            pltpu.SemaphoreType.DMA(((NP - 1) * B,)),
            pltpu.SemaphoreType.DMA(((NP - 1) * B,)),
            pltpu.SemaphoreType.DMA((4,)),
        ],
        compiler_params=pltpu.CompilerParams(
            collective_id=0,
            vmem_limit_bytes=63 * 1024 * 1024,
        ),
    )(x, Wdkv, Wuk, Wuv, Wq, Wqr, Wkr, Wo)
